# baseline (speedup 1.0000x reference)
import functools

import jax
import jax.numpy as jnp
from jax import lax
from jax.experimental import pallas as pl
from jax.experimental.pallas import tpu as pltpu
from jax.experimental.pallas import tpu_sc as plsc

N = 10000
NA = 10240
D_IN = 128
DH = 16
DOUT = 7

E_RAW = 320000
NTILES = 32
CB = 125
CHUNKS = 80
RPT = N // 16
RPTA = NA // 16
NBUF = 8
LOOK = 4
DEAD = N + 16

_MESH = dict(core_axis_name="c", subcore_axis_name="s")


def _redirect_edge0(idx_v, val):
    lane = lax.iota(jnp.int32, 16)
    row = idx_v[0, pl.ds(0, 16)]
    idx_v[0, pl.ds(0, 16)] = jnp.where(lane == 0, val, row)


@functools.partial(
    pl.kernel,
    out_type=jax.ShapeDtypeStruct((2, NA), jnp.float32),
    mesh=plsc.VectorSubcoreMesh(**_MESH),
    compiler_params=pltpu.CompilerParams(use_tc_tiling_on_sc=False),
    scratch_types=[
        pltpu.VMEM((CHUNKS, CB), jnp.int32),
        pltpu.VMEM((CB,), jnp.float32),
        pltpu.VMEM((RPTA,), jnp.float32),
        pltpu.VMEM_SHARED((NA,), jnp.float32),
        pltpu.SemaphoreType.DMA,
    ],
)
def _sc_deg(ei_hbm, out_hbm, dst_v, ones_v, zero_v, acc_sh, sem_s):
    c = lax.axis_index("c")
    s = lax.axis_index("s")
    wid = s * 2 + c

    for k in range(CB // 16 + 1):
        o = min(k * 16, CB - 16)
        ones_v[pl.ds(o, 16)] = jnp.full((16,), 1.0, jnp.float32)

    def zb(i, _):
        zero_v[pl.ds(i * 16, 16)] = jnp.zeros((16,), jnp.float32)
        return 0

    lax.fori_loop(0, RPTA // 16, zb, 0)
    pltpu.sync_copy(zero_v, acc_sh.at[pl.ds(s * RPTA, RPTA)])
    pltpu.sync_copy(ei_hbm.at[1, wid], dst_v)

    @pl.when(wid == 0)
    def _():
        _redirect_edge0(dst_v, DEAD)

    plsc.subcore_barrier()

    def body(i, _):
        pltpu.async_copy(ones_v, acc_sh.at[dst_v.at[i]], sem_s, add=True)
        return 0

    lax.fori_loop(0, CHUNKS, body, 0)

    def drain(i, _):
        pltpu.make_async_copy(ones_v, acc_sh.at[dst_v.at[i]], sem_s).wait()
        return 0

    lax.fori_loop(0, CHUNKS, drain, 0)
    plsc.subcore_barrier()
    pltpu.sync_copy(acc_sh.at[pl.ds(s * RPTA, RPTA)],
                    out_hbm.at[c, pl.ds(s * RPTA, RPTA)])


NP8 = NA * DH // 128
RP8 = RPTA * DH // 128


@functools.partial(
    pl.kernel,
    out_type=[jax.ShapeDtypeStruct((NA, DH), jnp.float32),
              jax.ShapeDtypeStruct((NA, DH), jnp.float32)],
    mesh=plsc.VectorSubcoreMesh(**_MESH),
    compiler_params=pltpu.CompilerParams(use_tc_tiling_on_sc=False),
    scratch_types=[
        pltpu.VMEM((CHUNKS, CB), jnp.int32),
        pltpu.VMEM((CHUNKS, CB), jnp.int32),
        pltpu.VMEM((NBUF, CB, DH), jnp.float32),
        pltpu.VMEM((RPTA, DH), jnp.float32),
        pltpu.VMEM_SHARED((NA, DH), jnp.float32),
        [pltpu.SemaphoreType.DMA] * NBUF,
        [pltpu.SemaphoreType.DMA] * NBUF,
    ],
)
def _sc_agg(hs_hbm, ei_hbm, out0_hbm, out1_hbm,
            src_v, dst_v, rows_v, zero_v, acc_sh, sem_g, sem_s):
    c = lax.axis_index("c")
    s = lax.axis_index("s")
    wid = s * 2 + c

    def zb(i, _):
        zero_v[i] = jnp.zeros((DH,), jnp.float32)
        return 0

    lax.fori_loop(0, RPTA, zb, 0)
    pltpu.sync_copy(zero_v, acc_sh.at[pl.ds(s * RPTA, RPTA)])
    pltpu.sync_copy(ei_hbm.at[0, wid], src_v)
    pltpu.sync_copy(ei_hbm.at[1, wid], dst_v)

    @pl.when(wid == 0)
    def _():
        _redirect_edge0(src_v, 0)
        _redirect_edge0(dst_v, DEAD)

    plsc.subcore_barrier()

    def gather(j, b):
        return pltpu.async_copy(hs_hbm.at[src_v.at[j]], rows_v.at[b],
                                sem_g[b])

    def scat(j, b, issue):
        if issue:
            return pltpu.async_copy(rows_v.at[b], acc_sh.at[dst_v.at[j]],
                                    sem_s[b], add=True)
        return pltpu.make_async_copy(rows_v.at[b], acc_sh.at[dst_v.at[j]],
                                     sem_s[b])

    for j in range(LOOK):
        gather(j, j % NBUF)

    def body(blk, _):
        base = blk * NBUF
        for p in range(NBUF):
            jj = base + p

            @pl.when(jj < CHUNKS)
            def _step():
                @pl.when(jj >= NBUF - LOOK)
                def _free():
                    scat(jj - (NBUF - LOOK), (p + LOOK) % NBUF,
                         False).wait()

                @pl.when(jj + LOOK < CHUNKS)
                def _prefetch():
                    gather(jj + LOOK, (p + LOOK) % NBUF)

                pltpu.make_async_copy(hs_hbm.at[src_v.at[jj]],
                                      rows_v.at[p], sem_g[p]).wait()
                scat(jj, p, True)

        return 0

    lax.fori_loop(0, (CHUNKS + NBUF - 1) // NBUF, body, 0)
    for j in range(max(0, CHUNKS - (NBUF - LOOK)), CHUNKS):
        scat(j, j % NBUF, False).wait()
    plsc.subcore_barrier()

    @pl.when(c == 0)
    def _w0():
        pltpu.sync_copy(acc_sh.at[pl.ds(s * RPTA, RPTA)],
                        out0_hbm.at[pl.ds(s * RPTA, RPTA)])

    @pl.when(c == 1)
    def _w1():
        pltpu.sync_copy(acc_sh.at[pl.ds(s * RPTA, RPTA)],
                        out1_hbm.at[pl.ds(s * RPTA, RPTA)])


BR = 1000


BP = 128


def _mm1_body(x_ref, w_ref, o_ref):
    outs = [
        jnp.dot(x_ref[:, s, :], w_ref[...],
                preferred_element_type=jnp.float32)
        for s in range(8)
    ]
    o_ref[...] = jnp.concatenate(outs, axis=1)


_mm1 = pl.pallas_call(
    _mm1_body,
    grid=(NP8 // BP,),
    in_specs=[
        pl.BlockSpec((BP, 8, D_IN), lambda i: (i, 0, 0)),
        pl.BlockSpec((D_IN, DH), lambda i: (0, 0)),
    ],
    out_specs=pl.BlockSpec((BP, 128), lambda i: (i, 0)),
    out_shape=jax.ShapeDtypeStruct((NP8, 128), jnp.float32),
)


def _mm2_body(h_ref, w_ref, o_ref):
    o_ref[...] = jnp.dot(h_ref[...], w_ref[...],
                         preferred_element_type=jnp.float32)


_mm2 = pl.pallas_call(
    _mm2_body,
    grid=(NP8 // BP,),
    in_specs=[
        pl.BlockSpec((BP, 128), lambda i: (i, 0)),
        pl.BlockSpec((128, 128), lambda i: (0, 0)),
    ],
    out_specs=pl.BlockSpec((BP, 128), lambda i: (i, 0)),
    out_shape=jax.ShapeDtypeStruct((NP8, 128), jnp.float32),
)


def _head_body(e_ref, wl_ref, bl_ref, logp_ref):
    h3 = jnp.maximum(e_ref[...], 0.0)
    logits = jnp.dot(h3, wl_ref[...],
                     preferred_element_type=jnp.float32) + bl_ref[...]
    m = jnp.max(logits, axis=1, keepdims=True)
    ex = jnp.exp(logits - m)
    lse = jnp.log(jnp.sum(ex, axis=1, keepdims=True)) + m
    logp_ref[...] = logits - lse


_head = pl.pallas_call(
    _head_body,
    grid=(N // BR,),
    in_specs=[
        pl.BlockSpec((BR, DH), lambda i: (i, 0)),
        pl.BlockSpec((DH, DOUT), lambda i: (0, 0)),
        pl.BlockSpec((1, DOUT), lambda i: (0, 0)),
    ],
    out_specs=pl.BlockSpec((BR, DOUT), lambda i: (i, 0)),
    out_shape=jax.ShapeDtypeStruct((N, DOUT), jnp.float32),
)


@jax.jit
def kernel(x, edge_index, W1, b1, W2, b2, Wl, bl):
    ei4 = edge_index.reshape(2, NTILES, CHUNKS, CB)

    degp = _sc_deg(ei4)
    dis = lax.rsqrt(degp[0] + degp[1] + 1.0)
    disp = jnp.broadcast_to(dis[:, None], (NA, DH)).reshape(NP8, 128)
    hs1p = _mm1(x.reshape(N // 8, 8, D_IN), W1) * disp
    p10, p11 = _sc_agg(hs1p.reshape(NA, DH), ei4)
    b1p = jnp.tile(b1, NA).reshape(NP8, 128)
    h2p = jnp.maximum(
        disp * (p10.reshape(NP8, 128) + p11.reshape(NP8, 128) + hs1p)
        + b1p, 0.0)
    hs2p = _mm2(h2p, jnp.kron(jnp.eye(8, dtype=jnp.float32), W2)) * disp
    p20, p21 = _sc_agg(hs2p.reshape(NA, DH), ei4)
    b2p = jnp.tile(b2, NA).reshape(NP8, 128)
    embp = disp * (p20.reshape(NP8, 128) + p21.reshape(NP8, 128) + hs2p) + b2p
    emb = embp.reshape(NA, DH)[:N]
    logp = _head(emb, Wl, bl.reshape(1, DOUT))
    return logp, emb


# speedup vs baseline: 1.0476x; 1.0476x over previous
import functools

import jax
import jax.numpy as jnp
from jax import lax
from jax.experimental import pallas as pl
from jax.experimental.pallas import tpu as pltpu
from jax.experimental.pallas import tpu_sc as plsc

N = 10000
NA = 10240
D_IN = 128
DH = 16
DOUT = 7

E_RAW = 320000
NTILES = 32
CB = 125
CHUNKS = 80
RPT = N // 16
RPTA = NA // 16
NBUF = 12
LOOK = 6
DEAD = N + 16

_MESH = dict(core_axis_name="c", subcore_axis_name="s")


def _redirect_edge0(idx_v, val):
    lane = lax.iota(jnp.int32, 16)
    row = idx_v[0, pl.ds(0, 16)]
    idx_v[0, pl.ds(0, 16)] = jnp.where(lane == 0, val, row)


@functools.partial(
    pl.kernel,
    out_type=jax.ShapeDtypeStruct((2, NA), jnp.float32),
    mesh=plsc.VectorSubcoreMesh(**_MESH),
    compiler_params=pltpu.CompilerParams(use_tc_tiling_on_sc=False),
    scratch_types=[
        pltpu.VMEM((CHUNKS, CB), jnp.int32),
        pltpu.VMEM((CB,), jnp.float32),
        pltpu.VMEM((RPTA,), jnp.float32),
        pltpu.VMEM_SHARED((NA,), jnp.float32),
        pltpu.SemaphoreType.DMA,
    ],
)
def _sc_deg(ei_hbm, out_hbm, dst_v, ones_v, zero_v, acc_sh, sem_s):
    c = lax.axis_index("c")
    s = lax.axis_index("s")
    wid = s * 2 + c

    for k in range(CB // 16 + 1):
        o = min(k * 16, CB - 16)
        ones_v[pl.ds(o, 16)] = jnp.full((16,), 1.0, jnp.float32)

    def zb(i, _):
        zero_v[pl.ds(i * 16, 16)] = jnp.zeros((16,), jnp.float32)
        return 0

    lax.fori_loop(0, RPTA // 16, zb, 0)
    pltpu.sync_copy(zero_v, acc_sh.at[pl.ds(s * RPTA, RPTA)])
    pltpu.sync_copy(ei_hbm.at[1, wid], dst_v)

    @pl.when(wid == 0)
    def _():
        _redirect_edge0(dst_v, DEAD)

    plsc.subcore_barrier()

    def body(i, _):
        pltpu.async_copy(ones_v, acc_sh.at[dst_v.at[i]], sem_s, add=True)
        return 0

    lax.fori_loop(0, CHUNKS, body, 0)

    def drain(i, _):
        pltpu.make_async_copy(ones_v, acc_sh.at[dst_v.at[i]], sem_s).wait()
        return 0

    lax.fori_loop(0, CHUNKS, drain, 0)
    plsc.subcore_barrier()
    pltpu.sync_copy(acc_sh.at[pl.ds(s * RPTA, RPTA)],
                    out_hbm.at[c, pl.ds(s * RPTA, RPTA)])


NP8 = NA * DH // 128
RP8 = RPTA * DH // 128


@functools.partial(
    pl.kernel,
    out_type=jax.ShapeDtypeStruct((2, NA, DH), jnp.float32),
    mesh=plsc.VectorSubcoreMesh(**_MESH),
    compiler_params=pltpu.CompilerParams(use_tc_tiling_on_sc=False),
    scratch_types=[
        pltpu.VMEM((CHUNKS, CB), jnp.int32),
        pltpu.VMEM((CHUNKS, CB), jnp.int32),
        pltpu.VMEM((NBUF, CB, DH), jnp.float32),
        pltpu.VMEM((RPTA, DH), jnp.float32),
        pltpu.VMEM_SHARED((NA, DH), jnp.float32),
        [pltpu.SemaphoreType.DMA] * NBUF,
        [pltpu.SemaphoreType.DMA] * NBUF,
    ],
)
def _sc_agg(hs_hbm, ei_hbm, out_hbm,
            src_v, dst_v, rows_v, zero_v, acc_sh, sem_g, sem_s):
    c = lax.axis_index("c")
    s = lax.axis_index("s")
    wid = s * 2 + c

    def zb(i, _):
        zero_v[i] = jnp.zeros((DH,), jnp.float32)
        return 0

    lax.fori_loop(0, RPTA, zb, 0)
    pltpu.sync_copy(zero_v, acc_sh.at[pl.ds(s * RPTA, RPTA)])
    pltpu.sync_copy(ei_hbm.at[0, wid], src_v)
    pltpu.sync_copy(ei_hbm.at[1, wid], dst_v)

    @pl.when(wid == 0)
    def _():
        _redirect_edge0(src_v, 0)
        _redirect_edge0(dst_v, DEAD)

    plsc.subcore_barrier()

    def gather(j, b):
        return pltpu.async_copy(hs_hbm.at[src_v.at[j]], rows_v.at[b],
                                sem_g[b])

    def scat(j, b, issue):
        if issue:
            return pltpu.async_copy(rows_v.at[b], acc_sh.at[dst_v.at[j]],
                                    sem_s[b], add=True)
        return pltpu.make_async_copy(rows_v.at[b], acc_sh.at[dst_v.at[j]],
                                     sem_s[b])

    for j in range(LOOK):
        gather(j, j % NBUF)

    def body(blk, _):
        base = blk * NBUF
        for p in range(NBUF):
            jj = base + p

            @pl.when(jj < CHUNKS)
            def _step():
                @pl.when(jj >= NBUF - LOOK)
                def _free():
                    scat(jj - (NBUF - LOOK), (p + LOOK) % NBUF,
                         False).wait()

                @pl.when(jj + LOOK < CHUNKS)
                def _prefetch():
                    gather(jj + LOOK, (p + LOOK) % NBUF)

                pltpu.make_async_copy(hs_hbm.at[src_v.at[jj]],
                                      rows_v.at[p], sem_g[p]).wait()
                scat(jj, p, True)

        return 0

    lax.fori_loop(0, (CHUNKS + NBUF - 1) // NBUF, body, 0)
    for j in range(max(0, CHUNKS - (NBUF - LOOK)), CHUNKS):
        scat(j, j % NBUF, False).wait()
    plsc.subcore_barrier()

    pltpu.sync_copy(acc_sh.at[pl.ds(s * RPTA, RPTA)],
                    out_hbm.at[c, pl.ds(s * RPTA, RPTA)])


BR = 1000


BP = 128


def _mm1_body(x_ref, w_ref, o_ref):
    outs = [
        jnp.dot(x_ref[:, s, :], w_ref[...],
                preferred_element_type=jnp.float32)
        for s in range(8)
    ]
    o_ref[...] = jnp.concatenate(outs, axis=1)


_mm1 = pl.pallas_call(
    _mm1_body,
    grid=(NP8 // BP,),
    in_specs=[
        pl.BlockSpec((BP, 8, D_IN), lambda i: (i, 0, 0)),
        pl.BlockSpec((D_IN, DH), lambda i: (0, 0)),
    ],
    out_specs=pl.BlockSpec((BP, 128), lambda i: (i, 0)),
    out_shape=jax.ShapeDtypeStruct((NP8, 128), jnp.float32),
)


def _mm2_body(h_ref, w_ref, o_ref):
    o_ref[...] = jnp.dot(h_ref[...], w_ref[...],
                         preferred_element_type=jnp.float32)


_mm2 = pl.pallas_call(
    _mm2_body,
    grid=(NP8 // BP,),
    in_specs=[
        pl.BlockSpec((BP, 128), lambda i: (i, 0)),
        pl.BlockSpec((128, 128), lambda i: (0, 0)),
    ],
    out_specs=pl.BlockSpec((BP, 128), lambda i: (i, 0)),
    out_shape=jax.ShapeDtypeStruct((NP8, 128), jnp.float32),
)


def _head_body(e_ref, wl_ref, bl_ref, logp_ref):
    h3 = jnp.maximum(e_ref[...], 0.0)
    logits = jnp.dot(h3, wl_ref[...],
                     preferred_element_type=jnp.float32) + bl_ref[...]
    m = jnp.max(logits, axis=1, keepdims=True)
    ex = jnp.exp(logits - m)
    lse = jnp.log(jnp.sum(ex, axis=1, keepdims=True)) + m
    logp_ref[...] = logits - lse


_head = pl.pallas_call(
    _head_body,
    grid=(N // BR,),
    in_specs=[
        pl.BlockSpec((BR, DH), lambda i: (i, 0)),
        pl.BlockSpec((DH, DOUT), lambda i: (0, 0)),
        pl.BlockSpec((1, DOUT), lambda i: (0, 0)),
    ],
    out_specs=pl.BlockSpec((BR, DOUT), lambda i: (i, 0)),
    out_shape=jax.ShapeDtypeStruct((N, DOUT), jnp.float32),
)


@jax.jit
def kernel(x, edge_index, W1, b1, W2, b2, Wl, bl):
    ei4 = edge_index.reshape(2, NTILES, CHUNKS, CB)

    degp = _sc_deg(ei4)
    dis = lax.rsqrt(degp[0] + degp[1] + 1.0)
    disp = jnp.broadcast_to(dis[:, None], (NA, DH)).reshape(NP8, 128)
    hs1p = _mm1(x.reshape(N // 8, 8, D_IN), W1) * disp
    a1 = _sc_agg(hs1p.reshape(NA, DH), ei4).reshape(2, NP8, 128)
    b1p = jnp.tile(b1, NA).reshape(NP8, 128)
    h2p = jnp.maximum(disp * (a1[0] + a1[1] + hs1p) + b1p, 0.0)
    hs2p = _mm2(h2p, jnp.kron(jnp.eye(8, dtype=jnp.float32), W2)) * disp
    a2 = _sc_agg(hs2p.reshape(NA, DH), ei4).reshape(2, NP8, 128)
    b2p = jnp.tile(b2, NA).reshape(NP8, 128)
    embp = disp * (a2[0] + a2[1] + hs2p) + b2p
    emb = embp.reshape(NA, DH)[:N]
    logp = _head(emb, Wl, bl.reshape(1, DOUT))
    return logp, emb
